# baseline (speedup 1.0000x reference)
import functools

import jax
import jax.numpy as jnp
from jax import lax
from jax.experimental import pallas as pl
from jax.experimental.pallas import tpu as pltpu
from jax.experimental.pallas import tpu_sc as plsc

VOCAB = 1_000_000
EMB = 64
ROWS = 4096 * 200
GRP = 128
NBUF = 4


def _make_gather():
    nc, ns = 2, 16
    nw = nc * ns
    rows_per_w = ROWS // nw
    ngrp = rows_per_w // GRP
    mesh = plsc.VectorSubcoreMesh(core_axis_name="c", subcore_axis_name="s")

    @functools.partial(
        pl.kernel,
        mesh=mesh,
        out_type=jax.ShapeDtypeStruct((ROWS, EMB), jnp.float32),
        scratch_types=[
            pltpu.VMEM((ngrp, GRP), jnp.int32),
            pltpu.VMEM((NBUF, GRP, EMB), jnp.float32),
        ] + [pltpu.SemaphoreType.DMA] * NBUF,
        compiler_params=pltpu.CompilerParams(use_tc_tiling_on_sc=False),
    )
    def gather_kernel(idx_hbm, table_hbm, out_hbm, idx_v, rows_v, *sems):
        wid = lax.axis_index("s") * nc + lax.axis_index("c")
        grp_base = wid * ngrp
        row_base = wid * rows_per_w

        pltpu.sync_copy(idx_hbm.at[pl.ds(grp_base, ngrp)], idx_v)

        for b in range(NBUF):
            pltpu.async_copy(
                table_hbm.at[idx_v.at[b]], rows_v.at[b], sems[b])

        @pl.loop(0, ngrp, step=NBUF)
        def _(g0):
            for b in range(NBUF):
                g = g0 + b
                pltpu.make_async_copy(
                    table_hbm.at[idx_v.at[g]], rows_v.at[b], sems[b]).wait()
                pltpu.sync_copy(
                    rows_v.at[b],
                    out_hbm.at[pl.ds(row_base + g * GRP, GRP)])
                nxt = g + NBUF

                @pl.when(nxt < ngrp)
                def _():
                    pltpu.async_copy(
                        table_hbm.at[idx_v.at[nxt]], rows_v.at[b], sems[b])

    return gather_kernel


_gather = _make_gather()


@jax.jit
def kernel(in_idx, token_emb):
    b, s = in_idx.shape
    idx2d = in_idx.astype(jnp.int32).reshape(ROWS // GRP, GRP)
    out = _gather(idx2d, token_emb)
    return out.reshape(b, s, EMB)
